# baseline (speedup 1.0000x reference)
import functools

import jax
import jax.numpy as jnp
from jax import lax
from jax.experimental import pallas as pl
from jax.experimental.pallas import tpu as pltpu
from jax.experimental.pallas import tpu_sc as plsc

N = 100000
F = 128
B = 50000
S = 10
E = 128

NC = 2
NS = 16
NW = NC * NS
CHUNK = 64
N_CHUNKS = 25
B_PER_W = CHUNK * N_CHUNKS
B_PAD = NW * B_PER_W
PAIR = 2 * B_PER_W
ROWS = CHUNK * S
HALF = CHUNK + 8
REGION = 2 * HALF

_mesh = plsc.VectorSubcoreMesh(core_axis_name="c", subcore_axis_name="s")


@functools.partial(
    pl.kernel,
    mesh=_mesh,
    out_type=(
        jax.ShapeDtypeStruct((B_PAD, F), jnp.float32),
        jax.ShapeDtypeStruct((B_PAD, F), jnp.float32),
    ),
    scratch_types=(
        pltpu.VMEM((CHUNK,), jnp.int32),
        pltpu.VMEM((CHUNK,), jnp.int32),
        pltpu.VMEM((CHUNK, F), jnp.float32),
        pltpu.VMEM((CHUNK, F), jnp.float32),
        pltpu.VMEM((ROWS, F), jnp.float32),
        pltpu.VMEM((ROWS,), jnp.int32),
        pltpu.VMEM((ROWS,), jnp.int32),
        pltpu.VMEM((ROWS,), jnp.int32),
        pltpu.VMEM((ROWS,), jnp.int32),
        pltpu.VMEM((CHUNK, F), jnp.float32),
        pltpu.VMEM_SHARED((NS * REGION, F), jnp.float32),
        pltpu.SemaphoreType.DMA,
        pltpu.SemaphoreType.DMA,
        pltpu.SemaphoreType.DMA,
        pltpu.SemaphoreType.DMA,
        pltpu.SemaphoreType.DMA,
        pltpu.SemaphoreType.DMA,
        pltpu.SemaphoreType.DMA,
    ),
)
def _sc_gather(fue_hbm, fap_hbm, nodes_hbm, nidx_hbm, dst_hbm, zeros_hbm,
               self_out, agg_out,
               nodes_a, nodes_b, self_a, self_b, neigh_v,
               nidx_a, nidx_b, dst_a, dst_b, zeros_v, agg_sp,
               sem_meta, sem_self, sem_so, sem_z, sem_rb, sem_sc, sem_ng):
    cid = lax.axis_index("c")
    sid = lax.axis_index("s")
    rb0 = sid * REGION
    my_off = sid * PAIR + cid * B_PER_W

    bufs = ((nodes_a, self_a, nidx_a, dst_a, 0),
            (nodes_b, self_b, nidx_b, dst_b, HALF))

    pltpu.sync_copy(zeros_hbm, zeros_v)

    def fire_meta(ci, bu):
        nodes_v, _, nidx_v, dst_v, _ = bu
        base = my_off + ci * CHUNK
        pltpu.async_copy(nodes_hbm.at[pl.ds(base, CHUNK)], nodes_v, sem_meta)
        for s in range(S):
            pltpu.async_copy(nidx_hbm.at[pl.ds(s * B_PAD + base, CHUNK)],
                             nidx_v.at[pl.ds(s * CHUNK, CHUNK)], sem_meta)
            pltpu.async_copy(dst_hbm.at[pl.ds(s * B_PAD + base, CHUNK)],
                             dst_v.at[pl.ds(s * CHUNK, CHUNK)], sem_meta)

    def drain_meta(bu):
        nodes_v, _, nidx_v, dst_v, _ = bu
        pltpu.make_async_copy(nodes_hbm.at[pl.ds(0, CHUNK)], nodes_v,
                              sem_meta).wait()
        for s in range(S):
            pltpu.make_async_copy(nidx_hbm.at[pl.ds(0, CHUNK)],
                                  nidx_v.at[pl.ds(s * CHUNK, CHUNK)],
                                  sem_meta).wait()
            pltpu.make_async_copy(dst_hbm.at[pl.ds(0, CHUNK)],
                                  dst_v.at[pl.ds(s * CHUNK, CHUNK)],
                                  sem_meta).wait()

    def chunk(ci, cur, nxt):
        nodes_v, self_v, nidx_v, dst_v, half = cur
        base = my_off + ci * CHUNK

        @pl.when(ci >= 2)
        def _():
            pltpu.make_async_copy(self_v, self_out.at[pl.ds(0, CHUNK)],
                                  sem_so).wait()

        drain_meta(cur)

        @pl.when(ci < N_CHUNKS - 1)
        def _():
            fire_meta(ci + 1, nxt)

        h_self = pltpu.async_copy(fue_hbm.at[nodes_v], self_v, sem_self)
        h_ng = pltpu.async_copy(fap_hbm.at[nidx_v], neigh_v, sem_ng)

        pltpu.make_async_copy(zeros_hbm, agg_sp.at[pl.ds(rb0, CHUNK)],
                              sem_z).wait()

        h_ng.wait()
        h_sc = pltpu.async_copy(neigh_v, agg_sp.at[dst_v], sem_sc, add=True)

        @pl.when(ci >= 1)
        def _():
            pltpu.make_async_copy(agg_sp.at[pl.ds(rb0, CHUNK)],
                                  agg_out.at[pl.ds(0, CHUNK)], sem_rb).wait()

        other_half = HALF - half
        @pl.when(ci < N_CHUNKS - 1)
        def _():
            pltpu.async_copy(zeros_v,
                             agg_sp.at[pl.ds(rb0 + other_half, CHUNK)], sem_z)

        h_sc.wait()
        pltpu.async_copy(agg_sp.at[pl.ds(rb0 + half, CHUNK)],
                         agg_out.at[pl.ds(base, CHUNK)], sem_rb)

        h_self.wait()
        pltpu.async_copy(self_v, self_out.at[pl.ds(base, CHUNK)], sem_so)

    fire_meta(0, bufs[0])
    pltpu.async_copy(zeros_v, agg_sp.at[pl.ds(rb0, CHUNK)], sem_z)
    chunk(0, bufs[0], bufs[1])

    def pair_body(k, _):
        ci = 1 + 2 * k
        chunk(ci, bufs[1], bufs[0])
        chunk(ci + 1, bufs[0], bufs[1])
        return ()

    lax.fori_loop(0, (N_CHUNKS - 1) // 2, pair_body, (), unroll=False)

    pltpu.make_async_copy(agg_sp.at[pl.ds(rb0, CHUNK)],
                          agg_out.at[pl.ds(0, CHUNK)], sem_rb).wait()
    for bu in bufs:
        pltpu.make_async_copy(bu[1], self_out.at[pl.ds(0, CHUNK)],
                              sem_so).wait()


BB = 2048


def _tc_body(self_ref, agg_ref, recip_ref, w1_ref, w2_ref, out_ref):
    a = agg_ref[...] * recip_ref[...]
    acc = lax.dot_general(w1_ref[...], self_ref[...],
                          (((1,), (1,)), ((), ())),
                          preferred_element_type=jnp.float32)
    acc = acc + lax.dot_general(w2_ref[...], a,
                                (((1,), (1,)), ((), ())),
                                preferred_element_type=jnp.float32)
    out_ref[...] = jnp.maximum(acc, 0.0)


def _tc_matmul(self_f, agg_f, recip, w1, w2):
    grid = (pl.cdiv(B, BB),)
    return pl.pallas_call(
        _tc_body,
        grid=grid,
        in_specs=[
            pl.BlockSpec((BB, F), lambda i: (i, 0)),
            pl.BlockSpec((BB, F), lambda i: (i, 0)),
            pl.BlockSpec((BB, 1), lambda i: (i, 0)),
            pl.BlockSpec((E, F), lambda i: (0, 0)),
            pl.BlockSpec((E, F), lambda i: (0, 0)),
        ],
        out_specs=pl.BlockSpec((E, BB), lambda i: (0, i)),
        out_shape=jax.ShapeDtypeStruct((E, B), jnp.float32),
    )(self_f, agg_f, recip, w1, w2)


def kernel(features_ue, features_ap, adj_lists_ue, adj_lists_ap, nodes, weight):
    nodes_p = jnp.pad(nodes, (0, B_PAD - B))
    adj_t = adj_lists_ue.T
    tn = jnp.take(adj_t, nodes_p, axis=1)
    eq = tn[:, None, :] == tn[None, :, :]
    earlier = jnp.tril(jnp.ones((S, S), dtype=bool), k=-1)
    is_dup = jnp.any(eq & earlier[:, :, None], axis=1)
    num_uniq = S - jnp.sum(is_dup.astype(jnp.float32), axis=0)
    recip_p = (1.0 / (num_uniq + 1.0))[:, None]

    g = jnp.arange(B_PAD, dtype=jnp.int32)
    sid = g // PAIR
    chunk_idx = (g % B_PER_W) // CHUNK
    rbase = sid * REGION + (chunk_idx % 2) * HALF
    dst = jnp.where(is_dup, (rbase + CHUNK)[None, :],
                    (rbase + (g % CHUNK))[None, :]).astype(jnp.int32)

    nidx = tn.reshape(S * B_PAD)
    dst = dst.reshape(S * B_PAD)
    zeros = jnp.zeros((CHUNK, F), jnp.float32)

    self_f, agg_f = _sc_gather(features_ue, features_ap, nodes_p,
                               nidx, dst, zeros)

    w1 = weight[:, :F]
    w2 = weight[:, F:]
    return _tc_matmul(self_f, agg_f, recip_p, w1, w2)


# speedup vs baseline: 1.0654x; 1.0654x over previous
import functools

import jax
import jax.numpy as jnp
from jax import lax
from jax.experimental import pallas as pl
from jax.experimental.pallas import tpu as pltpu
from jax.experimental.pallas import tpu_sc as plsc

N = 100000
F = 128
B = 50000
S = 10
E = 128

NC = 2
NS = 16
NW = NC * NS
CHUNK = 64
N_CHUNKS = 25
B_PER_W = CHUNK * N_CHUNKS
B_PAD = NW * B_PER_W
PAIR = 2 * B_PER_W
ROWS = CHUNK * S
NJ = 5
JW = ROWS // NJ
HALF = CHUNK + 8
REGION = 2 * HALF
NJ_PAD = 8
IDX_ROWS = NW * N_CHUNKS * NJ_PAD

_mesh = plsc.VectorSubcoreMesh(core_axis_name="c", subcore_axis_name="s")


@functools.partial(
    pl.kernel,
    mesh=_mesh,
    out_type=(
        jax.ShapeDtypeStruct((B_PAD, F), jnp.float32),
        jax.ShapeDtypeStruct((B_PAD, F), jnp.float32),
    ),
    scratch_types=(
        pltpu.VMEM((2, CHUNK), jnp.int32),
        pltpu.VMEM((2, CHUNK, F), jnp.float32),
        pltpu.VMEM((NJ, JW, F), jnp.float32),
        pltpu.VMEM((2, NJ_PAD, JW), jnp.int32),
        pltpu.VMEM((2, NJ_PAD, JW), jnp.int32),
        pltpu.VMEM((CHUNK, F), jnp.float32),
        pltpu.VMEM_SHARED((NS * REGION, F), jnp.float32),
        pltpu.SemaphoreType.DMA,
        pltpu.SemaphoreType.DMA,
        pltpu.SemaphoreType.DMA,
        pltpu.SemaphoreType.DMA,
        pltpu.SemaphoreType.DMA,
        pltpu.SemaphoreType.DMA,
        pltpu.SemaphoreType.DMA,
        pltpu.SemaphoreType.DMA,
        pltpu.SemaphoreType.DMA,
        pltpu.SemaphoreType.DMA,
        pltpu.SemaphoreType.DMA,
    ),
)
def _sc_gather(fue_hbm, fap_hbm, nodes_hbm, nidx_hbm, dst_hbm, zeros_hbm,
               self_out, agg_out,
               nodes_v, self_v, neigh_v, nidx_v, dst_v, zeros_v, agg_sp,
               sem_meta, sem_self, sem_so, sem_z, sem_rb, sem_sc,
               sng0, sng1, sng2, sng3, sng4):
    cid = lax.axis_index("c")
    sid = lax.axis_index("s")
    rb0 = sid * REGION
    sng = (sng0, sng1, sng2, sng3, sng4)
    my_off = sid * PAIR + cid * B_PER_W

    pltpu.sync_copy(zeros_hbm, zeros_v)

    def fire_meta(ci, pb):
        base = my_off + ci * CHUNK
        irow = (base // CHUNK) * NJ_PAD
        pltpu.async_copy(nodes_hbm.at[pl.ds(base, CHUNK)], nodes_v.at[pb],
                         sem_meta)
        pltpu.async_copy(nidx_hbm.at[pl.ds(irow, NJ_PAD)], nidx_v.at[pb],
                         sem_meta)
        pltpu.async_copy(dst_hbm.at[pl.ds(irow, NJ_PAD)], dst_v.at[pb],
                         sem_meta)

    def drain_meta(pb):
        pltpu.make_async_copy(nodes_hbm.at[pl.ds(0, CHUNK)], nodes_v.at[pb],
                              sem_meta).wait()
        pltpu.make_async_copy(nidx_hbm.at[pl.ds(0, NJ_PAD)], nidx_v.at[pb],
                              sem_meta).wait()
        pltpu.make_async_copy(dst_hbm.at[pl.ds(0, NJ_PAD)], dst_v.at[pb],
                              sem_meta).wait()

    fire_meta(0, 0)
    pltpu.async_copy(zeros_v, agg_sp.at[pl.ds(rb0, CHUNK)], sem_z)

    def chunk_body(ci, _):
        pb = ci % 2
        pb1 = 1 - pb
        base = my_off + ci * CHUNK

        @pl.when(ci >= 2)
        def _():
            pltpu.make_async_copy(self_v.at[pb],
                                  self_out.at[pl.ds(0, CHUNK)], sem_so).wait()

        drain_meta(pb)

        @pl.when(ci < N_CHUNKS - 1)
        def _():
            fire_meta(ci + 1, pb1)

        h_self = pltpu.async_copy(fue_hbm.at[nodes_v.at[pb]], self_v.at[pb],
                                  sem_self)
        h_ng = [pltpu.async_copy(fap_hbm.at[nidx_v.at[pb, j]], neigh_v.at[j],
                                 sng[j]) for j in range(NJ)]

        pltpu.make_async_copy(zeros_hbm, agg_sp.at[pl.ds(rb0, CHUNK)],
                              sem_z).wait()

        h_sc = []
        for j in range(NJ):
            h_ng[j].wait()
            h_sc.append(pltpu.async_copy(neigh_v.at[j],
                                         agg_sp.at[dst_v.at[pb, j]],
                                         sem_sc, add=True))

        @pl.when(ci >= 1)
        def _():
            pltpu.make_async_copy(agg_sp.at[pl.ds(rb0, CHUNK)],
                                  agg_out.at[pl.ds(0, CHUNK)], sem_rb).wait()

        @pl.when(ci < N_CHUNKS - 1)
        def _():
            pltpu.async_copy(zeros_v,
                             agg_sp.at[pl.ds(rb0 + pb1 * HALF, CHUNK)], sem_z)

        for h in h_sc:
            h.wait()
        pltpu.async_copy(agg_sp.at[pl.ds(rb0 + pb * HALF, CHUNK)],
                         agg_out.at[pl.ds(base, CHUNK)], sem_rb)

        h_self.wait()
        pltpu.async_copy(self_v.at[pb], self_out.at[pl.ds(base, CHUNK)],
                         sem_so)
        return ()

    lax.fori_loop(0, N_CHUNKS, chunk_body, (), unroll=False)

    pltpu.make_async_copy(agg_sp.at[pl.ds(rb0, CHUNK)],
                          agg_out.at[pl.ds(0, CHUNK)], sem_rb).wait()
    for pb in range(2):
        pltpu.make_async_copy(self_v.at[pb], self_out.at[pl.ds(0, CHUNK)],
                              sem_so).wait()


BB = 2048


def _tc_body(self_ref, agg_ref, recip_ref, w1_ref, w2_ref, out_ref):
    a = agg_ref[...] * recip_ref[...]
    acc = lax.dot_general(w1_ref[...], self_ref[...],
                          (((1,), (1,)), ((), ())),
                          preferred_element_type=jnp.float32)
    acc = acc + lax.dot_general(w2_ref[...], a,
                                (((1,), (1,)), ((), ())),
                                preferred_element_type=jnp.float32)
    out_ref[...] = jnp.maximum(acc, 0.0)


def _tc_matmul(self_f, agg_f, recip, w1, w2):
    grid = (pl.cdiv(B, BB),)
    return pl.pallas_call(
        _tc_body,
        grid=grid,
        in_specs=[
            pl.BlockSpec((BB, F), lambda i: (i, 0)),
            pl.BlockSpec((BB, F), lambda i: (i, 0)),
            pl.BlockSpec((BB, 1), lambda i: (i, 0)),
            pl.BlockSpec((E, F), lambda i: (0, 0)),
            pl.BlockSpec((E, F), lambda i: (0, 0)),
        ],
        out_specs=pl.BlockSpec((E, BB), lambda i: (0, i)),
        out_shape=jax.ShapeDtypeStruct((E, B), jnp.float32),
    )(self_f, agg_f, recip, w1, w2)


def kernel(features_ue, features_ap, adj_lists_ue, adj_lists_ap, nodes, weight):
    nodes_p = jnp.pad(nodes, (0, B_PAD - B))
    tn = jnp.take(adj_lists_ue, nodes_p, axis=0)
    eq = tn[:, :, None] == tn[:, None, :]
    earlier = jnp.tril(jnp.ones((S, S), dtype=bool), k=-1)
    is_dup = jnp.any(eq & earlier[None, :, :], axis=2)
    num_uniq = S - jnp.sum(is_dup.astype(jnp.float32), axis=1)
    recip_p = (1.0 / (num_uniq + 1.0))[:, None]

    g = jnp.arange(B_PAD, dtype=jnp.int32)
    sid = g // PAIR
    chunk_idx = (g % B_PER_W) // CHUNK
    rbase = sid * REGION + (chunk_idx % 2) * HALF
    dst = jnp.where(is_dup, (rbase + CHUNK)[:, None],
                    (rbase + (g % CHUNK))[:, None]).astype(jnp.int32)

    def _chunk_layout(x):
        x = x.reshape(NW * N_CHUNKS, NJ * JW)
        x = jnp.pad(x, ((0, 0), (0, (NJ_PAD - NJ) * JW)))
        return x.reshape(IDX_ROWS, JW)

    nidx = _chunk_layout(tn)
    dst = _chunk_layout(dst)
    zeros = jnp.zeros((CHUNK, F), jnp.float32)

    self_f, agg_f = _sc_gather(features_ue, features_ap, nodes_p,
                               nidx, dst, zeros)

    w1 = weight[:, :F]
    w2 = weight[:, F:]
    return _tc_matmul(self_f, agg_f, recip_p, w1, w2)


# speedup vs baseline: 1.0767x; 1.0106x over previous
import functools

import jax
import jax.numpy as jnp
from jax import lax
from jax.experimental import pallas as pl
from jax.experimental.pallas import tpu as pltpu
from jax.experimental.pallas import tpu_sc as plsc

N = 100000
F = 128
B = 50000
S = 10
E = 128

NC = 2
NS = 16
NW = NC * NS
CHUNK = 64
N_CHUNKS = 25
B_PER_W = CHUNK * N_CHUNKS
B_PAD = NW * B_PER_W
PAIR = 2 * B_PER_W
ROWS = CHUNK * S
NJ = 5
JW = ROWS // NJ
HALF = CHUNK + 8
REGION = 2 * HALF
NJ_PAD = 8
IDX_ROWS = NW * N_CHUNKS * NJ_PAD

_mesh = plsc.VectorSubcoreMesh(core_axis_name="c", subcore_axis_name="s")


@functools.partial(
    pl.kernel,
    mesh=_mesh,
    out_type=(
        jax.ShapeDtypeStruct((B_PAD, F), jnp.float32),
        jax.ShapeDtypeStruct((B_PAD, F), jnp.float32),
    ),
    scratch_types=(
        pltpu.VMEM((2, CHUNK), jnp.int32),
        pltpu.VMEM((2, CHUNK, F), jnp.float32),
        pltpu.VMEM((NJ, JW, F), jnp.float32),
        pltpu.VMEM((2, NJ_PAD, JW), jnp.int32),
        pltpu.VMEM((2, NJ_PAD, JW), jnp.int32),
        pltpu.VMEM((CHUNK, F), jnp.float32),
        pltpu.VMEM_SHARED((NS * REGION, F), jnp.float32),
        pltpu.SemaphoreType.DMA,
        pltpu.SemaphoreType.DMA,
        pltpu.SemaphoreType.DMA,
        pltpu.SemaphoreType.DMA,
        pltpu.SemaphoreType.DMA,
        pltpu.SemaphoreType.DMA,
        pltpu.SemaphoreType.DMA,
        pltpu.SemaphoreType.DMA,
        pltpu.SemaphoreType.DMA,
        pltpu.SemaphoreType.DMA,
        pltpu.SemaphoreType.DMA,
    ),
)
def _sc_gather(fue_hbm, fap_hbm, nodes_hbm, nidx_hbm, dst_hbm, zeros_hbm,
               self_out, agg_out,
               nodes_v, self_v, neigh_v, nidx_v, dst_v, zeros_v, agg_sp,
               sem_meta, sem_self, sem_so, sem_z, sem_rb, sem_sc,
               sng0, sng1, sng2, sng3, sng4):
    cid = lax.axis_index("c")
    sid = lax.axis_index("s")
    rb0 = sid * REGION
    sng = (sng0, sng1, sng2, sng3, sng4)
    my_off = sid * PAIR + cid * B_PER_W

    pltpu.sync_copy(zeros_hbm, zeros_v)

    def fire_meta(ci, pb):
        base = my_off + ci * CHUNK
        irow = (base // CHUNK) * NJ_PAD
        pltpu.async_copy(nodes_hbm.at[pl.ds(base, CHUNK)], nodes_v.at[pb],
                         sem_meta)
        pltpu.async_copy(nidx_hbm.at[pl.ds(irow, NJ_PAD)], nidx_v.at[pb],
                         sem_meta)
        pltpu.async_copy(dst_hbm.at[pl.ds(irow, NJ_PAD)], dst_v.at[pb],
                         sem_meta)

    def drain_meta(pb):
        pltpu.make_async_copy(nodes_hbm.at[pl.ds(0, CHUNK)], nodes_v.at[pb],
                              sem_meta).wait()
        pltpu.make_async_copy(nidx_hbm.at[pl.ds(0, NJ_PAD)], nidx_v.at[pb],
                              sem_meta).wait()
        pltpu.make_async_copy(dst_hbm.at[pl.ds(0, NJ_PAD)], dst_v.at[pb],
                              sem_meta).wait()

    fire_meta(0, 0)
    pltpu.async_copy(zeros_v, agg_sp.at[pl.ds(rb0, CHUNK)], sem_z)

    def chunk_body(ci, _):
        pb = ci % 2
        pb1 = 1 - pb
        base = my_off + ci * CHUNK

        @pl.when(ci >= 2)
        def _():
            pltpu.make_async_copy(self_v.at[pb],
                                  self_out.at[pl.ds(0, CHUNK)], sem_so).wait()

        drain_meta(pb)

        @pl.when(ci < N_CHUNKS - 1)
        def _():
            fire_meta(ci + 1, pb1)

        h_self = pltpu.async_copy(fue_hbm.at[nodes_v.at[pb]], self_v.at[pb],
                                  sem_self)
        h_ng = [pltpu.async_copy(fap_hbm.at[nidx_v.at[pb, j]], neigh_v.at[j],
                                 sng[j]) for j in range(NJ)]

        pltpu.make_async_copy(zeros_hbm, agg_sp.at[pl.ds(rb0, CHUNK)],
                              sem_z).wait()

        h_sc = []
        for j in range(NJ):
            h_ng[j].wait()
            h_sc.append(pltpu.async_copy(neigh_v.at[j],
                                         agg_sp.at[dst_v.at[pb, j]],
                                         sem_sc, add=True))

        @pl.when(ci >= 1)
        def _():
            pltpu.make_async_copy(agg_sp.at[pl.ds(rb0, CHUNK)],
                                  agg_out.at[pl.ds(0, CHUNK)], sem_rb).wait()

        @pl.when(ci < N_CHUNKS - 1)
        def _():
            pltpu.async_copy(zeros_v,
                             agg_sp.at[pl.ds(rb0 + pb1 * HALF, CHUNK)], sem_z)

        for h in h_sc:
            h.wait()
        pltpu.async_copy(agg_sp.at[pl.ds(rb0 + pb * HALF, CHUNK)],
                         agg_out.at[pl.ds(base, CHUNK)], sem_rb)

        h_self.wait()
        pltpu.async_copy(self_v.at[pb], self_out.at[pl.ds(base, CHUNK)],
                         sem_so)
        return ()

    lax.fori_loop(0, N_CHUNKS, chunk_body, (), unroll=False)

    pltpu.make_async_copy(agg_sp.at[pl.ds(rb0, CHUNK)],
                          agg_out.at[pl.ds(0, CHUNK)], sem_rb).wait()
    for pb in range(2):
        pltpu.make_async_copy(self_v.at[pb], self_out.at[pl.ds(0, CHUNK)],
                              sem_so).wait()


BB = 4096


def _tc_body(self_ref, agg_ref, recip_ref, w1_ref, w2_ref, out_ref):
    a = agg_ref[...] * recip_ref[...]
    acc = lax.dot_general(w1_ref[...], self_ref[...],
                          (((1,), (1,)), ((), ())),
                          preferred_element_type=jnp.float32)
    acc = acc + lax.dot_general(w2_ref[...], a,
                                (((1,), (1,)), ((), ())),
                                preferred_element_type=jnp.float32)
    out_ref[...] = jnp.maximum(acc, 0.0)


def _tc_matmul(self_f, agg_f, recip, w1, w2):
    grid = (pl.cdiv(B, BB),)
    return pl.pallas_call(
        _tc_body,
        grid=grid,
        in_specs=[
            pl.BlockSpec((BB, F), lambda i: (i, 0)),
            pl.BlockSpec((BB, F), lambda i: (i, 0)),
            pl.BlockSpec((BB, 1), lambda i: (i, 0)),
            pl.BlockSpec((E, F), lambda i: (0, 0)),
            pl.BlockSpec((E, F), lambda i: (0, 0)),
        ],
        out_specs=pl.BlockSpec((E, BB), lambda i: (0, i)),
        out_shape=jax.ShapeDtypeStruct((E, B), jnp.float32),
    )(self_f, agg_f, recip, w1, w2)


def kernel(features_ue, features_ap, adj_lists_ue, adj_lists_ap, nodes, weight):
    nodes_p = jnp.pad(nodes, (0, B_PAD - B))
    tn = jnp.take(adj_lists_ue, nodes_p, axis=0)
    eq = tn[:, :, None] == tn[:, None, :]
    earlier = jnp.tril(jnp.ones((S, S), dtype=bool), k=-1)
    is_dup = jnp.any(eq & earlier[None, :, :], axis=2)
    num_uniq = S - jnp.sum(is_dup.astype(jnp.float32), axis=1)
    recip_p = (1.0 / (num_uniq + 1.0))[:, None]

    g = jnp.arange(B_PAD, dtype=jnp.int32)
    sid = g // PAIR
    chunk_idx = (g % B_PER_W) // CHUNK
    rbase = sid * REGION + (chunk_idx % 2) * HALF
    dst = jnp.where(is_dup, (rbase + CHUNK)[:, None],
                    (rbase + (g % CHUNK))[:, None]).astype(jnp.int32)

    def _chunk_layout(x):
        x = x.reshape(NW * N_CHUNKS, NJ * JW)
        x = jnp.pad(x, ((0, 0), (0, (NJ_PAD - NJ) * JW)))
        return x.reshape(IDX_ROWS, JW)

    nidx = _chunk_layout(tn)
    dst = _chunk_layout(dst)
    zeros = jnp.zeros((CHUNK, F), jnp.float32)

    self_f, agg_f = _sc_gather(features_ue, features_ap, nodes_p,
                               nidx, dst, zeros)

    w1 = weight[:, :F]
    w2 = weight[:, F:]
    return _tc_matmul(self_f, agg_f, recip_p, w1, w2)
